# Initial kernel scaffold; baseline (speedup 1.0000x reference)
import jax
import jax.numpy as jnp
from jax.experimental import pallas as pl


def kernel(cost_matrix):
    raise NotImplementedError("write your pallas kernel here")


# speedup vs baseline: 43.2898x; 43.2898x over previous
import functools

import jax
import jax.numpy as jnp
from jax import lax
from jax.experimental import pallas as pl
from jax.experimental.pallas import tpu as pltpu
from jax.experimental.pallas import tpu_sc as plsc

R = 512
C = 2048
L = 16
RCH = R // L
CCH = C // L
N_TILES = 16
ROWS_PER_TILE = R // N_TILES
SLAB = ROWS_PER_TILE * C
BIG = 1 << 30

_i32 = jnp.int32
_f32 = jnp.float32


def _bfly_argmax(v, mi, iota):
    for d in (1, 2, 4, 8):
        pidx = iota ^ d
        ov = v.at[pidx].get(mode="promise_in_bounds")
        oi = mi.at[pidx].get(mode="promise_in_bounds")
        take = (ov > v) | ((ov == v) & (oi < mi))
        v = jnp.where(take, ov, v)
        mi = jnp.where(take, oi, mi)
    return v, mi


def _bfly_min(x, iota):
    for d in (1, 2, 4, 8):
        x = jnp.minimum(x, x.at[iota ^ d].get(mode="promise_in_bounds"))
    return x


def _lane0(x):
    return lax.squeeze(lax.slice(x, (0,), (1,)), (0,))


def _lsa_body(cost_hbm, rowind_hbm, colind_hbm,
              rowbuf, colmask, rbv, rbc, rbv_loc, rbc_loc,
              outr, outc, mat_s, rbv_s, rbc_s, sem_a):
    sid = lax.axis_index("s")
    iota = lax.iota(_i32, L)
    lane0 = iota == 0
    negvec = jnp.full((L,), -jnp.inf, _f32)
    zvec = jnp.zeros((L,), _i32)
    bigvec = jnp.full((L,), BIG, _i32)

    r0 = sid * ROWS_PER_TILE
    base = r0 * C

    pltpu.async_copy(cost_hbm.at[pl.ds(base, SLAB)],
                     mat_s.at[pl.ds(base, SLAB)], sem_a).wait()

    def row_body(k, carry):
        pltpu.sync_copy(mat_s.at[pl.ds(base + k * C, C)], rowbuf)
        def chunk(q, mc):
            m, mi = mc
            v = rowbuf[pl.ds(q * L, L)]
            colv = q * L + iota
            better = v > m
            return (jnp.where(better, v, m), jnp.where(better, colv, mi))
        m, mi = lax.fori_loop(0, CCH, chunk, (negvec, zvec), unroll=4)
        mxv, civ = _bfly_argmax(m, mi, iota)
        kvec = jnp.full((L,), k, _i32)
        plsc.store_scatter(rbv_loc, [kvec], mxv, mask=lane0)
        plsc.store_scatter(rbc_loc, [kvec], civ, mask=lane0)
        return carry

    lax.fori_loop(0, ROWS_PER_TILE, row_body, 0)

    pltpu.sync_copy(rbv_loc, rbv_s.at[pl.ds(r0, ROWS_PER_TILE)])
    pltpu.sync_copy(rbc_loc, rbc_s.at[pl.ds(r0, ROWS_PER_TILE)])
    plsc.subcore_barrier()

    @pl.when(sid == 0)
    def _greedy():
        pltpu.sync_copy(rbv_s, rbv)
        pltpu.sync_copy(rbc_s, rbc)

        def zchunk(q, carry):
            colmask[pl.ds(q * L, L)] = jnp.zeros((L,), _f32)
            return carry
        lax.fori_loop(0, CCH, zchunk, 0)

        def step(i, carry):
            def achunk(q, mc):
                m, mi = mc
                v = rbv[pl.ds(q * L, L)]
                rowv = q * L + iota
                better = v > m
                return (jnp.where(better, v, m), jnp.where(better, rowv, mi))
            m, mi = lax.fori_loop(0, RCH, achunk, (negvec, zvec), unroll=4)
            _, rvec = _bfly_argmax(m, mi, iota)
            cvec = plsc.load_gather(rbc, [rvec])
            plsc.store_scatter(outc, [rvec], cvec, mask=lane0)
            plsc.store_scatter(rbv, [rvec], negvec, mask=lane0)
            plsc.store_scatter(colmask, [cvec], negvec, mask=lane0)

            def dscan():
                def dchunk(q, jminv):
                    cb = rbc[pl.ds(q * L, L)]
                    vb = rbv[pl.ds(q * L, L)]
                    match = (cb == cvec) & (vb > negvec)
                    rowv = q * L + iota
                    return jnp.where(match, jnp.minimum(jminv, rowv), jminv)
                jminv = lax.fori_loop(0, RCH, dchunk, bigvec, unroll=4)
                return _lane0(_bfly_min(jminv, iota))

            def rcond(j):
                return j < BIG

            def rbody(j):
                pltpu.sync_copy(mat_s.at[pl.ds(j * C, C)], rowbuf)
                def rchunk(q, mc):
                    m2, mi2 = mc
                    v = rowbuf[pl.ds(q * L, L)] + colmask[pl.ds(q * L, L)]
                    colv = q * L + iota
                    better = v > m2
                    return (jnp.where(better, v, m2),
                            jnp.where(better, colv, mi2))
                m2, mi2 = lax.fori_loop(0, CCH, rchunk, (negvec, zvec), unroll=4)
                mx2v, c2v = _bfly_argmax(m2, mi2, iota)
                jvec = jnp.full((L,), j, _i32)
                plsc.store_scatter(rbv, [jvec], mx2v, mask=lane0)
                plsc.store_scatter(rbc, [jvec], c2v, mask=lane0)
                return dscan()

            lax.while_loop(rcond, rbody, dscan())
            return carry

        lax.fori_loop(0, R, step, 0)

        def ochunk(q, carry):
            outr[pl.ds(q * L, L)] = q * L + iota
            return carry
        lax.fori_loop(0, RCH, ochunk, 0)
        pltpu.sync_copy(outr, rowind_hbm)
        pltpu.sync_copy(outc, colind_hbm)


_lsa = pl.kernel(
    _lsa_body,
    out_type=(jax.ShapeDtypeStruct((R,), _i32),
              jax.ShapeDtypeStruct((R,), _i32)),
    mesh=plsc.VectorSubcoreMesh(core_axis_name="c", subcore_axis_name="s",
                                num_cores=1, num_subcores=N_TILES),
    compiler_params=pltpu.CompilerParams(needs_layout_passes=False),
    scratch_types=[
        pltpu.VMEM((C,), _f32),
        pltpu.VMEM((C,), _f32),
        pltpu.VMEM((R,), _f32),
        pltpu.VMEM((R,), _i32),
        pltpu.VMEM((ROWS_PER_TILE,), _f32),
        pltpu.VMEM((ROWS_PER_TILE,), _i32),
        pltpu.VMEM((R,), _i32),
        pltpu.VMEM((R,), _i32),
        pltpu.VMEM_SHARED((R * C,), _f32),
        pltpu.VMEM_SHARED((R,), _f32),
        pltpu.VMEM_SHARED((R,), _i32),
        pltpu.SemaphoreType.DMA,
    ],
)


def kernel(cost_matrix):
    flat = cost_matrix.reshape(-1)
    row_ind, col_ind = _lsa(flat)
    return row_ind, col_ind


# speedup vs baseline: 55.5476x; 1.2832x over previous
import functools

import jax
import jax.numpy as jnp
from jax import lax
from jax.experimental import pallas as pl
from jax.experimental.pallas import tpu as pltpu
from jax.experimental.pallas import tpu_sc as plsc

R = 512
C = 2048
L = 16
RCH = R // L
CCH = C // L
N_TILES = 16
ROWS_PER_TILE = R // N_TILES
SLAB = ROWS_PER_TILE * C
BIG = 1 << 30

_i32 = jnp.int32
_f32 = jnp.float32


def _bfly_argmax(v, mi, iota):
    for d in (1, 2, 4, 8):
        pidx = iota ^ d
        ov = v.at[pidx].get(mode="promise_in_bounds")
        oi = mi.at[pidx].get(mode="promise_in_bounds")
        take = (ov > v) | ((ov == v) & (oi < mi))
        v = jnp.where(take, ov, v)
        mi = jnp.where(take, oi, mi)
    return v, mi


def _bfly_min(x, iota):
    for d in (1, 2, 4, 8):
        x = jnp.minimum(x, x.at[iota ^ d].get(mode="promise_in_bounds"))
    return x


def _bfly_max(x, iota):
    for d in (1, 2, 4, 8):
        x = jnp.maximum(x, x.at[iota ^ d].get(mode="promise_in_bounds"))
    return x


def _lane0(x):
    return lax.squeeze(lax.slice(x, (0,), (1,)), (0,))


def _lsa_body(cost_hbm, rowind_hbm, colind_hbm,
              rowbuf, colmask, cnt, rbv, rbc, cm, rbv_loc, rbc_loc,
              outr, outc, mat_s, rbv_s, rbc_s, sem_a):
    sid = lax.axis_index("s")
    iota = lax.iota(_i32, L)
    lane0 = iota == 0
    negvec = jnp.full((L,), -jnp.inf, _f32)
    zvec = jnp.zeros((L,), _i32)
    bigvec = jnp.full((L,), BIG, _i32)

    r0 = sid * ROWS_PER_TILE
    base = r0 * C

    pltpu.async_copy(cost_hbm.at[pl.ds(base, SLAB)],
                     mat_s.at[pl.ds(base, SLAB)], sem_a).wait()

    def row_body(k, carry):
        pltpu.sync_copy(mat_s.at[pl.ds(base + k * C, C)], rowbuf)
        def chunk(q, mc):
            m, mi = mc
            v = rowbuf[pl.ds(q * L, L)]
            colv = q * L + iota
            better = v > m
            return (jnp.where(better, v, m), jnp.where(better, colv, mi))
        m, mi = lax.fori_loop(0, CCH, chunk, (negvec, zvec), unroll=4)
        mxv, civ = _bfly_argmax(m, mi, iota)
        kvec = jnp.full((L,), k, _i32)
        plsc.store_scatter(rbv_loc, [kvec], mxv, mask=lane0)
        plsc.store_scatter(rbc_loc, [kvec], civ, mask=lane0)
        return carry

    lax.fori_loop(0, ROWS_PER_TILE, row_body, 0)

    pltpu.sync_copy(rbv_loc, rbv_s.at[pl.ds(r0, ROWS_PER_TILE)])
    pltpu.sync_copy(rbc_loc, rbc_s.at[pl.ds(r0, ROWS_PER_TILE)])
    plsc.subcore_barrier()

    @pl.when(sid == 0)
    def _greedy():
        pltpu.sync_copy(rbv_s, rbv)
        pltpu.sync_copy(rbc_s, rbc)
        ones = jnp.full((L,), 1, _i32)
        mones = jnp.full((L,), -1, _i32)

        def zchunk(q, carry):
            colmask[pl.ds(q * L, L)] = jnp.zeros((L,), _f32)
            cnt[pl.ds(q * L, L)] = jnp.zeros((L,), _i32)
            return carry
        lax.fori_loop(0, CCH, zchunk, 0)

        def cmchunk(q, carry):
            hm = _bfly_max(rbv[pl.ds(q * L, L)], iota)
            plsc.store_scatter(cm, [jnp.full((L,), q, _i32)], hm, mask=lane0)
            return carry
        lax.fori_loop(0, RCH, cmchunk, 0)

        def cntchunk(q, carry):
            cb = rbc[pl.ds(q * L, L)]
            for l in range(L):
                plsc.addupdate_scatter(cnt, [cb], ones, mask=iota == l)
            return carry
        lax.fori_loop(0, RCH, cntchunk, 0)

        def step(i, carry):
            v1 = cm[pl.ds(0, L)]
            v2 = cm[pl.ds(L, L)]
            b = v2 > v1
            m32 = jnp.where(b, v2, v1)
            q32 = jnp.where(b, iota + L, iota)
            _, qv = _bfly_argmax(m32, q32, iota)
            qs = _lane0(qv)
            chunk = rbv[pl.ds(qs * L, L)]
            _, rvec = _bfly_argmax(chunk, qs * L + iota, iota)
            cvec = plsc.load_gather(rbc, [rvec])
            plsc.store_scatter(outc, [rvec], cvec, mask=lane0)
            plsc.store_scatter(rbv, [rvec], negvec, mask=lane0)
            plsc.store_scatter(colmask, [cvec], negvec, mask=lane0)
            plsc.addupdate_scatter(cnt, [cvec], mones, mask=lane0)
            hm = _bfly_max(rbv[pl.ds(qs * L, L)], iota)
            plsc.store_scatter(cm, [qv], hm, mask=lane0)

            def dscan():
                def dchunk(q, jminv):
                    cb = rbc[pl.ds(q * L, L)]
                    vb = rbv[pl.ds(q * L, L)]
                    match = (cb == cvec) & (vb > negvec)
                    rowv = q * L + iota
                    return jnp.where(match, jnp.minimum(jminv, rowv), jminv)
                jminv = lax.fori_loop(0, RCH, dchunk, bigvec, unroll=4)
                return _lane0(_bfly_min(jminv, iota))

            def rcond(cc):
                return cc > 0

            def rbody(cc):
                j = dscan()
                pltpu.sync_copy(mat_s.at[pl.ds(j * C, C)], rowbuf)
                def rchunk(q, mc):
                    m2, mi2 = mc
                    v = rowbuf[pl.ds(q * L, L)] + colmask[pl.ds(q * L, L)]
                    colv = q * L + iota
                    better = v > m2
                    return (jnp.where(better, v, m2),
                            jnp.where(better, colv, mi2))
                m2, mi2 = lax.fori_loop(0, CCH, rchunk, (negvec, zvec), unroll=4)
                mx2v, c2v = _bfly_argmax(m2, mi2, iota)
                jvec = jnp.full((L,), j, _i32)
                plsc.store_scatter(rbv, [jvec], mx2v, mask=lane0)
                plsc.store_scatter(rbc, [jvec], c2v, mask=lane0)
                plsc.addupdate_scatter(cnt, [c2v], ones, mask=lane0)
                jq = lax.shift_right_logical(j, 4)
                hm2 = _bfly_max(rbv[pl.ds(jq * L, L)], iota)
                plsc.store_scatter(cm, [jnp.full((L,), jq, _i32)], hm2,
                                   mask=lane0)
                return cc - 1

            cc0 = _lane0(plsc.load_gather(cnt, [cvec]))
            lax.while_loop(rcond, rbody, cc0)
            return carry

        lax.fori_loop(0, R, step, 0)

        def ochunk(q, carry):
            outr[pl.ds(q * L, L)] = q * L + iota
            return carry
        lax.fori_loop(0, RCH, ochunk, 0)
        pltpu.sync_copy(outr, rowind_hbm)
        pltpu.sync_copy(outc, colind_hbm)


_lsa = pl.kernel(
    _lsa_body,
    out_type=(jax.ShapeDtypeStruct((R,), _i32),
              jax.ShapeDtypeStruct((R,), _i32)),
    mesh=plsc.VectorSubcoreMesh(core_axis_name="c", subcore_axis_name="s",
                                num_cores=1, num_subcores=N_TILES),
    compiler_params=pltpu.CompilerParams(needs_layout_passes=False),
    scratch_types=[
        pltpu.VMEM((C,), _f32),
        pltpu.VMEM((C,), _f32),
        pltpu.VMEM((C,), _i32),
        pltpu.VMEM((R,), _f32),
        pltpu.VMEM((R,), _i32),
        pltpu.VMEM((RCH,), _f32),
        pltpu.VMEM((ROWS_PER_TILE,), _f32),
        pltpu.VMEM((ROWS_PER_TILE,), _i32),
        pltpu.VMEM((R,), _i32),
        pltpu.VMEM((R,), _i32),
        pltpu.VMEM_SHARED((R * C,), _f32),
        pltpu.VMEM_SHARED((R,), _f32),
        pltpu.VMEM_SHARED((R,), _i32),
        pltpu.SemaphoreType.DMA,
    ],
)


def kernel(cost_matrix):
    flat = cost_matrix.reshape(-1)
    row_ind, col_ind = _lsa(flat)
    return row_ind, col_ind


# speedup vs baseline: 58.1489x; 1.0468x over previous
import functools

import jax
import jax.numpy as jnp
from jax import lax
from jax.experimental import pallas as pl
from jax.experimental.pallas import tpu as pltpu
from jax.experimental.pallas import tpu_sc as plsc

R = 512
C = 2048
L = 16
RCH = R // L
CCH = C // L
N_TILES = 16
ROWS_PER_TILE = R // N_TILES
SLAB = ROWS_PER_TILE * C
BIG = 1 << 30

_i32 = jnp.int32
_f32 = jnp.float32


def _bfly_argmax(v, mi, iota):
    for d in (1, 2, 4, 8):
        pidx = iota ^ d
        ov = v.at[pidx].get(mode="promise_in_bounds")
        oi = mi.at[pidx].get(mode="promise_in_bounds")
        take = (ov > v) | ((ov == v) & (oi < mi))
        v = jnp.where(take, ov, v)
        mi = jnp.where(take, oi, mi)
    return v, mi


def _bfly_min(x, iota):
    for d in (1, 2, 4, 8):
        x = jnp.minimum(x, x.at[iota ^ d].get(mode="promise_in_bounds"))
    return x


def _bfly_max(x, iota):
    for d in (1, 2, 4, 8):
        x = jnp.maximum(x, x.at[iota ^ d].get(mode="promise_in_bounds"))
    return x


def _lane0(x):
    return lax.squeeze(lax.slice(x, (0,), (1,)), (0,))


def _lsa_body(cost_hbm, rowind_hbm, colind_hbm,
              rowbuf, rowbuf_b, colmask, cnt, rbv, rbc, cm, rbv_loc, rbc_loc,
              outr, outc, mat_s, rbv_s, rbc_s, sem_a, sem_b):
    sid = lax.axis_index("s")
    iota = lax.iota(_i32, L)
    lane0 = iota == 0
    negvec = jnp.full((L,), -jnp.inf, _f32)
    zvec = jnp.zeros((L,), _i32)
    bigvec = jnp.full((L,), BIG, _i32)

    r0 = sid * ROWS_PER_TILE
    base = r0 * C

    pltpu.async_copy(cost_hbm.at[pl.ds(base, SLAB)],
                     mat_s.at[pl.ds(base, SLAB)], sem_a).wait()

    def scan_row(buf, k):
        def chunk(q, mc):
            m, mi = mc
            v = buf[pl.ds(q * L, L)]
            colv = q * L + iota
            better = v > m
            return (jnp.where(better, v, m), jnp.where(better, colv, mi))
        m, mi = lax.fori_loop(0, CCH, chunk, (negvec, zvec), unroll=8)
        mxv, civ = _bfly_argmax(m, mi, iota)
        kvec = jnp.full((L,), k, _i32)
        plsc.store_scatter(rbv_loc, [kvec], mxv, mask=lane0)
        plsc.store_scatter(rbc_loc, [kvec], civ, mask=lane0)

    pltpu.async_copy(mat_s.at[pl.ds(base, C)], rowbuf, sem_a)

    def row_pair(k2, carry):
        ka = 2 * k2
        pltpu.make_async_copy(mat_s.at[pl.ds(base + ka * C, C)],
                              rowbuf, sem_a).wait()
        pltpu.async_copy(mat_s.at[pl.ds(base + (ka + 1) * C, C)],
                         rowbuf_b, sem_b)
        scan_row(rowbuf, ka)
        pltpu.make_async_copy(mat_s.at[pl.ds(base + (ka + 1) * C, C)],
                              rowbuf_b, sem_b).wait()
        @pl.when(k2 < ROWS_PER_TILE // 2 - 1)
        def _():
            pltpu.async_copy(mat_s.at[pl.ds(base + (ka + 2) * C, C)],
                             rowbuf, sem_a)
        scan_row(rowbuf_b, ka + 1)
        return carry

    lax.fori_loop(0, ROWS_PER_TILE // 2, row_pair, 0)

    pltpu.sync_copy(rbv_loc, rbv_s.at[pl.ds(r0, ROWS_PER_TILE)])
    pltpu.sync_copy(rbc_loc, rbc_s.at[pl.ds(r0, ROWS_PER_TILE)])
    plsc.subcore_barrier()

    @pl.when(sid == 0)
    def _greedy():
        pltpu.sync_copy(rbv_s, rbv)
        pltpu.sync_copy(rbc_s, rbc)
        ones = jnp.full((L,), 1, _i32)

        def zchunk(q, carry):
            colmask[pl.ds(q * L, L)] = jnp.zeros((L,), _f32)
            cnt[pl.ds(q * L, L)] = jnp.zeros((L,), _i32)
            return carry
        lax.fori_loop(0, CCH, zchunk, 0)

        def cmchunk(q, carry):
            hm = _bfly_max(rbv[pl.ds(q * L, L)], iota)
            plsc.store_scatter(cm, [jnp.full((L,), q, _i32)], hm, mask=lane0)
            return carry
        lax.fori_loop(0, RCH, cmchunk, 0)

        def cntchunk(q, carry):
            cb = rbc[pl.ds(q * L, L)]
            for l in range(L):
                plsc.addupdate_scatter(cnt, [cb], ones, mask=iota == l)
            return carry
        lax.fori_loop(0, RCH, cntchunk, 0)

        def step(i, carry):
            v1 = cm[pl.ds(0, L)]
            v2 = cm[pl.ds(L, L)]
            b = v2 > v1
            m32 = jnp.where(b, v2, v1)
            q32 = jnp.where(b, iota + L, iota)
            _, qv = _bfly_argmax(m32, q32, iota)
            qs = _lane0(qv)
            chunk = rbv[pl.ds(qs * L, L)]
            _, rvec = _bfly_argmax(chunk, qs * L + iota, iota)
            cvec = plsc.load_gather(rbc, [rvec])
            plsc.store_scatter(outc, [rvec], cvec, mask=lane0)
            plsc.store_scatter(rbv, [rvec], negvec, mask=lane0)
            plsc.store_scatter(colmask, [cvec], negvec, mask=lane0)
            hm = _bfly_max(rbv[pl.ds(qs * L, L)], iota)
            plsc.store_scatter(cm, [qv], hm, mask=lane0)

            def dscan():
                def dchunk(q, jminv):
                    cb = rbc[pl.ds(q * L, L)]
                    vb = rbv[pl.ds(q * L, L)]
                    match = (cb == cvec) & (vb > negvec)
                    rowv = q * L + iota
                    return jnp.where(match, jnp.minimum(jminv, rowv), jminv)
                jminv = lax.fori_loop(0, RCH, dchunk, bigvec, unroll=8)
                return _lane0(_bfly_min(jminv, iota))

            def rcond(cc):
                return cc > 0

            def rbody(cc):
                j = dscan()
                pltpu.sync_copy(mat_s.at[pl.ds(j * C, C)], rowbuf)
                def rchunk(q, mc):
                    m2, mi2 = mc
                    v = rowbuf[pl.ds(q * L, L)] + colmask[pl.ds(q * L, L)]
                    colv = q * L + iota
                    better = v > m2
                    return (jnp.where(better, v, m2),
                            jnp.where(better, colv, mi2))
                m2, mi2 = lax.fori_loop(0, CCH, rchunk, (negvec, zvec), unroll=8)
                mx2v, c2v = _bfly_argmax(m2, mi2, iota)
                jvec = jnp.full((L,), j, _i32)
                plsc.store_scatter(rbv, [jvec], mx2v, mask=lane0)
                plsc.store_scatter(rbc, [jvec], c2v, mask=lane0)
                plsc.addupdate_scatter(cnt, [c2v], ones, mask=lane0)
                jq = lax.shift_right_logical(j, 4)
                hm2 = _bfly_max(rbv[pl.ds(jq * L, L)], iota)
                plsc.store_scatter(cm, [jnp.full((L,), jq, _i32)], hm2,
                                   mask=lane0)
                return cc - 1

            cc0 = _lane0(plsc.load_gather(cnt, [cvec])) - 1
            lax.while_loop(rcond, rbody, cc0)
            return carry

        lax.fori_loop(0, R, step, 0)

        def ochunk(q, carry):
            outr[pl.ds(q * L, L)] = q * L + iota
            return carry
        lax.fori_loop(0, RCH, ochunk, 0)
        pltpu.sync_copy(outr, rowind_hbm)
        pltpu.sync_copy(outc, colind_hbm)


_lsa = pl.kernel(
    _lsa_body,
    out_type=(jax.ShapeDtypeStruct((R,), _i32),
              jax.ShapeDtypeStruct((R,), _i32)),
    mesh=plsc.VectorSubcoreMesh(core_axis_name="c", subcore_axis_name="s",
                                num_cores=1, num_subcores=N_TILES),
    compiler_params=pltpu.CompilerParams(needs_layout_passes=False),
    scratch_types=[
        pltpu.VMEM((C,), _f32),
        pltpu.VMEM((C,), _f32),
        pltpu.VMEM((C,), _f32),
        pltpu.VMEM((C,), _i32),
        pltpu.VMEM((R,), _f32),
        pltpu.VMEM((R,), _i32),
        pltpu.VMEM((RCH,), _f32),
        pltpu.VMEM((ROWS_PER_TILE,), _f32),
        pltpu.VMEM((ROWS_PER_TILE,), _i32),
        pltpu.VMEM((R,), _i32),
        pltpu.VMEM((R,), _i32),
        pltpu.VMEM_SHARED((R * C,), _f32),
        pltpu.VMEM_SHARED((R,), _f32),
        pltpu.VMEM_SHARED((R,), _i32),
        pltpu.SemaphoreType.DMA,
        pltpu.SemaphoreType.DMA,
    ],
)


def kernel(cost_matrix):
    flat = cost_matrix.reshape(-1)
    row_ind, col_ind = _lsa(flat)
    return row_ind, col_ind


# speedup vs baseline: 63.0833x; 1.0849x over previous
import functools

import jax
import jax.numpy as jnp
from jax import lax
from jax.experimental import pallas as pl
from jax.experimental.pallas import tpu as pltpu
from jax.experimental.pallas import tpu_sc as plsc

R = 512
C = 2048
L = 16
RCH = R // L
CCH = C // L
N_TILES = 16
ROWS_PER_TILE = R // N_TILES
SLAB = ROWS_PER_TILE * C
BIG = 1 << 30

_i32 = jnp.int32
_f32 = jnp.float32


def _bfly_argmax(v, mi, iota):
    for d in (1, 2, 4, 8):
        pidx = iota ^ d
        ov = v.at[pidx].get(mode="promise_in_bounds")
        oi = mi.at[pidx].get(mode="promise_in_bounds")
        take = (ov > v) | ((ov == v) & (oi < mi))
        v = jnp.where(take, ov, v)
        mi = jnp.where(take, oi, mi)
    return v, mi


def _bfly_min(x, iota):
    for d in (1, 2, 4, 8):
        x = jnp.minimum(x, x.at[iota ^ d].get(mode="promise_in_bounds"))
    return x


def _bfly_max(x, iota):
    for d in (1, 2, 4, 8):
        x = jnp.maximum(x, x.at[iota ^ d].get(mode="promise_in_bounds"))
    return x


def _lane0(x):
    return lax.squeeze(lax.slice(x, (0,), (1,)), (0,))


def _bfly_argmax_top2(v, mi, iota):
    m1, i1, m2 = v, mi, jnp.full((L,), -jnp.inf, _f32)
    for d in (1, 2, 4, 8):
        pidx = iota ^ d
        o1 = m1.at[pidx].get(mode="promise_in_bounds")
        oi = i1.at[pidx].get(mode="promise_in_bounds")
        o2 = m2.at[pidx].get(mode="promise_in_bounds")
        take = (o1 > m1) | ((o1 == m1) & (oi < i1))
        nm1 = jnp.where(take, o1, m1)
        ni1 = jnp.where(take, oi, i1)
        nm2 = jnp.maximum(jnp.minimum(m1, o1), jnp.maximum(m2, o2))
        m1, i1, m2 = nm1, ni1, nm2
    return i1, m2


def _lsa_body(cost_hbm, rowind_hbm, colind_hbm,
              rowbuf, rowbuf_b, colmask, cnt, rbv, rbc, cm, rbv_loc, rbc_loc,
              outr, outc, mat_s, rbv_s, rbc_s, sem_a, sem_b):
    sid = lax.axis_index("s")
    iota = lax.iota(_i32, L)
    lane0 = iota == 0
    negvec = jnp.full((L,), -jnp.inf, _f32)
    zvec = jnp.zeros((L,), _i32)
    bigvec = jnp.full((L,), BIG, _i32)

    r0 = sid * ROWS_PER_TILE
    base = r0 * C

    pltpu.async_copy(cost_hbm.at[pl.ds(base, SLAB)],
                     mat_s.at[pl.ds(base, SLAB)], sem_a).wait()

    def scan_row(buf, k):
        def chunk(q, mc):
            m, mi = mc
            v = buf[pl.ds(q * L, L)]
            colv = q * L + iota
            better = v > m
            return (jnp.where(better, v, m), jnp.where(better, colv, mi))
        m, mi = lax.fori_loop(0, CCH, chunk, (negvec, zvec), unroll=8)
        mxv, civ = _bfly_argmax(m, mi, iota)
        kvec = jnp.full((L,), k, _i32)
        plsc.store_scatter(rbv_loc, [kvec], mxv, mask=lane0)
        plsc.store_scatter(rbc_loc, [kvec], civ, mask=lane0)

    pltpu.async_copy(mat_s.at[pl.ds(base, C)], rowbuf, sem_a)

    def row_pair(k2, carry):
        ka = 2 * k2
        pltpu.make_async_copy(mat_s.at[pl.ds(base + ka * C, C)],
                              rowbuf, sem_a).wait()
        pltpu.async_copy(mat_s.at[pl.ds(base + (ka + 1) * C, C)],
                         rowbuf_b, sem_b)
        scan_row(rowbuf, ka)
        pltpu.make_async_copy(mat_s.at[pl.ds(base + (ka + 1) * C, C)],
                              rowbuf_b, sem_b).wait()
        @pl.when(k2 < ROWS_PER_TILE // 2 - 1)
        def _():
            pltpu.async_copy(mat_s.at[pl.ds(base + (ka + 2) * C, C)],
                             rowbuf, sem_a)
        scan_row(rowbuf_b, ka + 1)
        return carry

    lax.fori_loop(0, ROWS_PER_TILE // 2, row_pair, 0)

    pltpu.sync_copy(rbv_loc, rbv_s.at[pl.ds(r0, ROWS_PER_TILE)])
    pltpu.sync_copy(rbc_loc, rbc_s.at[pl.ds(r0, ROWS_PER_TILE)])
    plsc.subcore_barrier()

    @pl.when(sid == 0)
    def _greedy():
        pltpu.sync_copy(rbv_s, rbv)
        pltpu.sync_copy(rbc_s, rbc)
        ones = jnp.full((L,), 1, _i32)

        def zchunk(q, carry):
            colmask[pl.ds(q * L, L)] = jnp.zeros((L,), _f32)
            cnt[pl.ds(q * L, L)] = jnp.zeros((L,), _i32)
            return carry
        lax.fori_loop(0, CCH, zchunk, 0)

        def cmchunk(q, carry):
            hm = _bfly_max(rbv[pl.ds(q * L, L)], iota)
            plsc.store_scatter(cm, [jnp.full((L,), q, _i32)], hm, mask=lane0)
            return carry
        lax.fori_loop(0, RCH, cmchunk, 0)

        def cntchunk(q, carry):
            cb = rbc[pl.ds(q * L, L)]
            for l in range(L):
                plsc.addupdate_scatter(cnt, [cb], ones, mask=iota == l)
            return carry
        lax.fori_loop(0, RCH, cntchunk, 0)

        def step(i, carry):
            v1 = cm[pl.ds(0, L)]
            v2 = cm[pl.ds(L, L)]
            b = v2 > v1
            m32 = jnp.where(b, v2, v1)
            q32 = jnp.where(b, iota + L, iota)
            _, qv = _bfly_argmax(m32, q32, iota)
            rowsv = qv * L + iota
            chunk = plsc.load_gather(rbv, [rowsv])
            rvec, hm = _bfly_argmax_top2(chunk, rowsv, iota)
            cvec = plsc.load_gather(rbc, [rvec])
            plsc.store_scatter(outc, [rvec], cvec, mask=lane0)
            plsc.store_scatter(rbv, [rvec], negvec, mask=lane0)
            plsc.store_scatter(colmask, [cvec], negvec, mask=lane0)
            plsc.store_scatter(cm, [qv], hm, mask=lane0)

            def dscan():
                def dchunk(q, jminv):
                    cb = rbc[pl.ds(q * L, L)]
                    vb = rbv[pl.ds(q * L, L)]
                    match = (cb == cvec) & (vb > negvec)
                    rowv = q * L + iota
                    return jnp.where(match, jnp.minimum(jminv, rowv), jminv)
                jminv = lax.fori_loop(0, RCH, dchunk, bigvec, unroll=8)
                return _lane0(_bfly_min(jminv, iota))

            def rcond(cc):
                return cc > 0

            def rbody(cc):
                j = dscan()
                pltpu.sync_copy(mat_s.at[pl.ds(j * C, C)], rowbuf)
                def rchunk(q, mc):
                    m2, mi2 = mc
                    v = rowbuf[pl.ds(q * L, L)] + colmask[pl.ds(q * L, L)]
                    colv = q * L + iota
                    better = v > m2
                    return (jnp.where(better, v, m2),
                            jnp.where(better, colv, mi2))
                m2, mi2 = lax.fori_loop(0, CCH, rchunk, (negvec, zvec), unroll=8)
                mx2v, c2v = _bfly_argmax(m2, mi2, iota)
                jvec = jnp.full((L,), j, _i32)
                plsc.store_scatter(rbv, [jvec], mx2v, mask=lane0)
                plsc.store_scatter(rbc, [jvec], c2v, mask=lane0)
                plsc.addupdate_scatter(cnt, [c2v], ones, mask=lane0)
                jq = lax.shift_right_logical(j, 4)
                hm2 = _bfly_max(rbv[pl.ds(jq * L, L)], iota)
                plsc.store_scatter(cm, [jnp.full((L,), jq, _i32)], hm2,
                                   mask=lane0)
                return cc - 1

            cc0 = _lane0(plsc.load_gather(cnt, [cvec])) - 1
            lax.while_loop(rcond, rbody, cc0)
            return carry

        lax.fori_loop(0, R, step, 0)

        def ochunk(q, carry):
            outr[pl.ds(q * L, L)] = q * L + iota
            return carry
        lax.fori_loop(0, RCH, ochunk, 0)
        pltpu.sync_copy(outr, rowind_hbm)
        pltpu.sync_copy(outc, colind_hbm)


_lsa = pl.kernel(
    _lsa_body,
    out_type=(jax.ShapeDtypeStruct((R,), _i32),
              jax.ShapeDtypeStruct((R,), _i32)),
    mesh=plsc.VectorSubcoreMesh(core_axis_name="c", subcore_axis_name="s",
                                num_cores=1, num_subcores=N_TILES),
    compiler_params=pltpu.CompilerParams(needs_layout_passes=False),
    scratch_types=[
        pltpu.VMEM((C,), _f32),
        pltpu.VMEM((C,), _f32),
        pltpu.VMEM((C,), _f32),
        pltpu.VMEM((C,), _i32),
        pltpu.VMEM((R,), _f32),
        pltpu.VMEM((R,), _i32),
        pltpu.VMEM((RCH,), _f32),
        pltpu.VMEM((ROWS_PER_TILE,), _f32),
        pltpu.VMEM((ROWS_PER_TILE,), _i32),
        pltpu.VMEM((R,), _i32),
        pltpu.VMEM((R,), _i32),
        pltpu.VMEM_SHARED((R * C,), _f32),
        pltpu.VMEM_SHARED((R,), _f32),
        pltpu.VMEM_SHARED((R,), _i32),
        pltpu.SemaphoreType.DMA,
        pltpu.SemaphoreType.DMA,
    ],
)


def kernel(cost_matrix):
    flat = cost_matrix.reshape(-1)
    row_ind, col_ind = _lsa(flat)
    return row_ind, col_ind


# speedup vs baseline: 69.1315x; 1.0959x over previous
import functools

import jax
import jax.numpy as jnp
from jax import lax
from jax.experimental import pallas as pl
from jax.experimental.pallas import tpu as pltpu
from jax.experimental.pallas import tpu_sc as plsc

R = 512
C = 2048
L = 16
RCH = R // L
CCH = C // L
N_TILES = 16
ROWS_PER_TILE = R // N_TILES
SLAB = ROWS_PER_TILE * C
BIG = 1 << 30

_i32 = jnp.int32
_f32 = jnp.float32


def _bfly_argmax(v, mi, iota):
    for d in (1, 2, 4, 8):
        pidx = iota ^ d
        ov = v.at[pidx].get(mode="promise_in_bounds")
        oi = mi.at[pidx].get(mode="promise_in_bounds")
        take = (ov > v) | ((ov == v) & (oi < mi))
        v = jnp.where(take, ov, v)
        mi = jnp.where(take, oi, mi)
    return v, mi


def _bfly_min(x, iota):
    for d in (1, 2, 4, 8):
        x = jnp.minimum(x, x.at[iota ^ d].get(mode="promise_in_bounds"))
    return x


def _bfly_max(x, iota):
    for d in (1, 2, 4, 8):
        x = jnp.maximum(x, x.at[iota ^ d].get(mode="promise_in_bounds"))
    return x


def _lane0(x):
    return lax.squeeze(lax.slice(x, (0,), (1,)), (0,))


def _bfly_argmax2_full(v, vi, v2, vi2, iota):
    m1, i1, m2, i2 = v, vi, v2, vi2
    for d in (1, 2, 4, 8):
        pidx = iota ^ d
        o1 = m1.at[pidx].get(mode="promise_in_bounds")
        oi1 = i1.at[pidx].get(mode="promise_in_bounds")
        o2 = m2.at[pidx].get(mode="promise_in_bounds")
        oi2 = i2.at[pidx].get(mode="promise_in_bounds")
        t1 = (o1 > m1) | ((o1 == m1) & (oi1 < i1))
        w1 = jnp.where(t1, o1, m1)
        wi1 = jnp.where(t1, oi1, i1)
        l1 = jnp.where(t1, m1, o1)
        li1 = jnp.where(t1, i1, oi1)
        t2 = (o2 > m2) | ((o2 == m2) & (oi2 < i2))
        w2 = jnp.where(t2, o2, m2)
        wi2 = jnp.where(t2, oi2, i2)
        t3 = (l1 > w2) | ((l1 == w2) & (li1 < wi2))
        m1, i1 = w1, wi1
        m2 = jnp.where(t3, l1, w2)
        i2 = jnp.where(t3, li1, wi2)
    return i1, m2, i2


def _bfly_argmax_top2(v, mi, iota):
    m1, i1, m2 = v, mi, jnp.full((L,), -jnp.inf, _f32)
    for d in (1, 2, 4, 8):
        pidx = iota ^ d
        o1 = m1.at[pidx].get(mode="promise_in_bounds")
        oi = i1.at[pidx].get(mode="promise_in_bounds")
        o2 = m2.at[pidx].get(mode="promise_in_bounds")
        take = (o1 > m1) | ((o1 == m1) & (oi < i1))
        nm1 = jnp.where(take, o1, m1)
        ni1 = jnp.where(take, oi, i1)
        nm2 = jnp.maximum(jnp.minimum(m1, o1), jnp.maximum(m2, o2))
        m1, i1, m2 = nm1, ni1, nm2
    return i1, m2


def _lsa_body(cost_hbm, rowind_hbm, colind_hbm,
              rowbuf, rowbuf_b, colmask, cnt, rbv, rbc, cm, rbv_loc, rbc_loc,
              outr, outc, mat_s, rbv_s, rbc_s, sem_a, sem_b):
    sid = lax.axis_index("s")
    iota = lax.iota(_i32, L)
    lane0 = iota == 0
    negvec = jnp.full((L,), -jnp.inf, _f32)
    zvec = jnp.zeros((L,), _i32)
    bigvec = jnp.full((L,), BIG, _i32)

    r0 = sid * ROWS_PER_TILE
    base = r0 * C

    pltpu.async_copy(cost_hbm.at[pl.ds(base, SLAB)],
                     mat_s.at[pl.ds(base, SLAB)], sem_a).wait()

    def scan_row(buf, k):
        def chunk(q, mc):
            m, mi = mc
            v = buf[pl.ds(q * L, L)]
            colv = q * L + iota
            better = v > m
            return (jnp.where(better, v, m), jnp.where(better, colv, mi))
        m, mi = lax.fori_loop(0, CCH, chunk, (negvec, zvec), unroll=8)
        mxv, civ = _bfly_argmax(m, mi, iota)
        kvec = jnp.full((L,), k, _i32)
        plsc.store_scatter(rbv_loc, [kvec], mxv, mask=lane0)
        plsc.store_scatter(rbc_loc, [kvec], civ, mask=lane0)

    pltpu.async_copy(mat_s.at[pl.ds(base, C)], rowbuf, sem_a)

    def row_pair(k2, carry):
        ka = 2 * k2
        pltpu.make_async_copy(mat_s.at[pl.ds(base + ka * C, C)],
                              rowbuf, sem_a).wait()
        pltpu.async_copy(mat_s.at[pl.ds(base + (ka + 1) * C, C)],
                         rowbuf_b, sem_b)
        scan_row(rowbuf, ka)
        pltpu.make_async_copy(mat_s.at[pl.ds(base + (ka + 1) * C, C)],
                              rowbuf_b, sem_b).wait()
        @pl.when(k2 < ROWS_PER_TILE // 2 - 1)
        def _():
            pltpu.async_copy(mat_s.at[pl.ds(base + (ka + 2) * C, C)],
                             rowbuf, sem_a)
        scan_row(rowbuf_b, ka + 1)
        return carry

    lax.fori_loop(0, ROWS_PER_TILE // 2, row_pair, 0)

    pltpu.sync_copy(rbv_loc, rbv_s.at[pl.ds(r0, ROWS_PER_TILE)])
    pltpu.sync_copy(rbc_loc, rbc_s.at[pl.ds(r0, ROWS_PER_TILE)])
    plsc.subcore_barrier()

    @pl.when(sid == 0)
    def _greedy():
        pltpu.sync_copy(rbv_s, rbv)
        pltpu.sync_copy(rbc_s, rbc)
        ones = jnp.full((L,), 1, _i32)

        def zchunk(q, carry):
            colmask[pl.ds(q * L, L)] = jnp.zeros((L,), _f32)
            cnt[pl.ds(q * L, L)] = jnp.zeros((L,), _i32)
            return carry
        lax.fori_loop(0, CCH, zchunk, 0)

        def cmchunk(q, carry):
            hm = _bfly_max(rbv[pl.ds(q * L, L)], iota)
            plsc.store_scatter(cm, [jnp.full((L,), q, _i32)], hm, mask=lane0)
            return carry
        lax.fori_loop(0, RCH, cmchunk, 0)

        def cntchunk(q, carry):
            cb = rbc[pl.ds(q * L, L)]
            for l in range(L):
                plsc.addupdate_scatter(cnt, [cb], ones, mask=iota == l)
            return carry
        lax.fori_loop(0, RCH, cntchunk, 0)

        def step(assigned):
            va = cm[pl.ds(0, L)]
            vb = cm[pl.ds(L, L)]
            tb = vb > va
            m1 = jnp.where(tb, vb, va)
            q1 = jnp.where(tb, iota + L, iota)
            m2 = jnp.where(tb, va, vb)
            q2 = jnp.where(tb, iota, iota + L)
            q1v, g2v, q2v = _bfly_argmax2_full(m1, q1, m2, q2, iota)
            rows1 = q1v * L + iota
            chunk1 = plsc.load_gather(rbv, [rows1])
            r1vec, hm1 = _bfly_argmax_top2(chunk1, rows1, iota)
            rows2 = q2v * L + iota
            chunk2 = plsc.load_gather(rbv, [rows2])
            r2vec, hm2 = _bfly_argmax_top2(chunk2, rows2, iota)
            c1vec = plsc.load_gather(rbc, [r1vec])
            c2vec = plsc.load_gather(rbc, [r2vec])
            plsc.store_scatter(outc, [r1vec], c1vec, mask=lane0)
            plsc.store_scatter(rbv, [r1vec], negvec, mask=lane0)
            plsc.store_scatter(colmask, [c1vec], negvec, mask=lane0)
            plsc.store_scatter(cm, [q1v], hm1, mask=lane0)
            cn1 = _lane0(plsc.load_gather(cnt, [c1vec])) - 1
            ok2v = jnp.where((g2v > hm1) & (c2vec != c1vec), ones, zvec)
            cond2 = (_lane0(ok2v) > 0) & (cn1 == 0)

            @pl.when(cond2)
            def _assign2():
                plsc.store_scatter(outc, [r2vec], c2vec, mask=lane0)
                plsc.store_scatter(rbv, [r2vec], negvec, mask=lane0)
                plsc.store_scatter(colmask, [c2vec], negvec, mask=lane0)
                plsc.store_scatter(cm, [q2v], hm2, mask=lane0)

            cond2v = jnp.full((L,), cond2, jnp.bool_)
            ctgt = jnp.where(cond2v, c2vec, c1vec)

            def dscan():
                def dchunk(q, jminv):
                    cb = rbc[pl.ds(q * L, L)]
                    vb_ = rbv[pl.ds(q * L, L)]
                    match = (cb == ctgt) & (vb_ > negvec)
                    rowv = q * L + iota
                    return jnp.where(match, jnp.minimum(jminv, rowv), jminv)
                jminv = lax.fori_loop(0, RCH, dchunk, bigvec, unroll=8)
                return _lane0(_bfly_min(jminv, iota))

            def rcond(cc):
                return cc > 0

            def rbody(cc):
                j = dscan()
                pltpu.sync_copy(mat_s.at[pl.ds(j * C, C)], rowbuf)
                def rchunk(q, mc):
                    mr, mir = mc
                    v = rowbuf[pl.ds(q * L, L)] + colmask[pl.ds(q * L, L)]
                    colv = q * L + iota
                    better = v > mr
                    return (jnp.where(better, v, mr),
                            jnp.where(better, colv, mir))
                mr, mir = lax.fori_loop(0, CCH, rchunk, (negvec, zvec), unroll=8)
                mxv, cnv = _bfly_argmax(mr, mir, iota)
                jvec = jnp.full((L,), j, _i32)
                plsc.store_scatter(rbv, [jvec], mxv, mask=lane0)
                plsc.store_scatter(rbc, [jvec], cnv, mask=lane0)
                plsc.addupdate_scatter(cnt, [cnv], ones, mask=lane0)
                jq = lax.shift_right_logical(j, 4)
                hmr = _bfly_max(rbv[pl.ds(jq * L, L)], iota)
                plsc.store_scatter(cm, [jnp.full((L,), jq, _i32)], hmr,
                                   mask=lane0)
                return cc - 1

            cc0 = _lane0(plsc.load_gather(cnt, [ctgt])) - 1
            lax.while_loop(rcond, rbody, cc0)
            return assigned + 1 + cond2.astype(_i32)

        lax.while_loop(lambda a: a < R, step, 0)

        def ochunk(q, carry):
            outr[pl.ds(q * L, L)] = q * L + iota
            return carry
        lax.fori_loop(0, RCH, ochunk, 0)
        pltpu.sync_copy(outr, rowind_hbm)
        pltpu.sync_copy(outc, colind_hbm)


_lsa = pl.kernel(
    _lsa_body,
    out_type=(jax.ShapeDtypeStruct((R,), _i32),
              jax.ShapeDtypeStruct((R,), _i32)),
    mesh=plsc.VectorSubcoreMesh(core_axis_name="c", subcore_axis_name="s",
                                num_cores=1, num_subcores=N_TILES),
    compiler_params=pltpu.CompilerParams(needs_layout_passes=False),
    scratch_types=[
        pltpu.VMEM((C,), _f32),
        pltpu.VMEM((C,), _f32),
        pltpu.VMEM((C,), _f32),
        pltpu.VMEM((C,), _i32),
        pltpu.VMEM((R,), _f32),
        pltpu.VMEM((R,), _i32),
        pltpu.VMEM((RCH,), _f32),
        pltpu.VMEM((ROWS_PER_TILE,), _f32),
        pltpu.VMEM((ROWS_PER_TILE,), _i32),
        pltpu.VMEM((R,), _i32),
        pltpu.VMEM((R,), _i32),
        pltpu.VMEM_SHARED((R * C,), _f32),
        pltpu.VMEM_SHARED((R,), _f32),
        pltpu.VMEM_SHARED((R,), _i32),
        pltpu.SemaphoreType.DMA,
        pltpu.SemaphoreType.DMA,
    ],
)


def kernel(cost_matrix):
    flat = cost_matrix.reshape(-1)
    row_ind, col_ind = _lsa(flat)
    return row_ind, col_ind


# speedup vs baseline: 70.9423x; 1.0262x over previous
import functools

import jax
import jax.numpy as jnp
from jax import lax
from jax.experimental import pallas as pl
from jax.experimental.pallas import tpu as pltpu
from jax.experimental.pallas import tpu_sc as plsc

R = 512
C = 2048
L = 16
RCH = R // L
CCH = C // L
N_TILES = 16
ROWS_PER_TILE = R // N_TILES
SLAB = ROWS_PER_TILE * C
BIG = 1 << 30

_i32 = jnp.int32
_f32 = jnp.float32


def _bfly_argmax(v, mi, iota):
    for d in (1, 2, 4, 8):
        pidx = iota ^ d
        ov = v.at[pidx].get(mode="promise_in_bounds")
        oi = mi.at[pidx].get(mode="promise_in_bounds")
        take = (ov > v) | ((ov == v) & (oi < mi))
        v = jnp.where(take, ov, v)
        mi = jnp.where(take, oi, mi)
    return v, mi


def _bfly_min(x, iota):
    for d in (1, 2, 4, 8):
        x = jnp.minimum(x, x.at[iota ^ d].get(mode="promise_in_bounds"))
    return x


def _bfly_max(x, iota):
    for d in (1, 2, 4, 8):
        x = jnp.maximum(x, x.at[iota ^ d].get(mode="promise_in_bounds"))
    return x


def _lane0(x):
    return lax.squeeze(lax.slice(x, (0,), (1,)), (0,))


def _bfly_argmax2_full(v, vi, v2, vi2, iota):
    m1, i1, m2, i2 = v, vi, v2, vi2
    for d in (1, 2, 4, 8):
        pidx = iota ^ d
        o1 = m1.at[pidx].get(mode="promise_in_bounds")
        oi1 = i1.at[pidx].get(mode="promise_in_bounds")
        o2 = m2.at[pidx].get(mode="promise_in_bounds")
        oi2 = i2.at[pidx].get(mode="promise_in_bounds")
        t1 = (o1 > m1) | ((o1 == m1) & (oi1 < i1))
        w1 = jnp.where(t1, o1, m1)
        wi1 = jnp.where(t1, oi1, i1)
        l1 = jnp.where(t1, m1, o1)
        li1 = jnp.where(t1, i1, oi1)
        t2 = (o2 > m2) | ((o2 == m2) & (oi2 < i2))
        w2 = jnp.where(t2, o2, m2)
        wi2 = jnp.where(t2, oi2, i2)
        t3 = (l1 > w2) | ((l1 == w2) & (li1 < wi2))
        m1, i1 = w1, wi1
        m2 = jnp.where(t3, l1, w2)
        i2 = jnp.where(t3, li1, wi2)
    return i1, m2, i2


def _bfly_argmax_top2(v, mi, iota):
    m1, i1, m2 = v, mi, jnp.full((L,), -jnp.inf, _f32)
    for d in (1, 2, 4, 8):
        pidx = iota ^ d
        o1 = m1.at[pidx].get(mode="promise_in_bounds")
        oi = i1.at[pidx].get(mode="promise_in_bounds")
        o2 = m2.at[pidx].get(mode="promise_in_bounds")
        take = (o1 > m1) | ((o1 == m1) & (oi < i1))
        nm1 = jnp.where(take, o1, m1)
        ni1 = jnp.where(take, oi, i1)
        nm2 = jnp.maximum(jnp.minimum(m1, o1), jnp.maximum(m2, o2))
        m1, i1, m2 = nm1, ni1, nm2
    return i1, m2


def _lsa_body(cost_hbm, rowind_hbm, colind_hbm,
              rowbuf, rowbuf_b, rowbuf_c, rowbuf_d, colmask, cnt, rbv, rbc,
              cm, rbv_loc, rbc_loc, outr, outc, mat_s, rbv_s, rbc_s,
              sem_a, sem_b, sem_c, sem_d, sem_e):
    sid = lax.axis_index("s")
    iota = lax.iota(_i32, L)
    lane0 = iota == 0
    negvec = jnp.full((L,), -jnp.inf, _f32)
    zvec = jnp.zeros((L,), _i32)
    bigvec = jnp.full((L,), BIG, _i32)

    r0 = sid * ROWS_PER_TILE
    base = r0 * C

    pltpu.async_copy(cost_hbm.at[pl.ds(base, SLAB)],
                     mat_s.at[pl.ds(base, SLAB)], sem_e)

    def scan_row(buf, k):
        def chunk(q, mc):
            m, mi = mc
            v = buf[pl.ds(q * L, L)]
            colv = q * L + iota
            better = v > m
            return (jnp.where(better, v, m), jnp.where(better, colv, mi))
        m, mi = lax.fori_loop(0, CCH, chunk, (negvec, zvec), unroll=8)
        mxv, civ = _bfly_argmax(m, mi, iota)
        kvec = jnp.full((L,), k, _i32)
        plsc.store_scatter(rbv_loc, [kvec], mxv, mask=lane0)
        plsc.store_scatter(rbc_loc, [kvec], civ, mask=lane0)

    bufs = (rowbuf, rowbuf_b, rowbuf_c, rowbuf_d)
    sems = (sem_a, sem_b, sem_c, sem_d)
    for idx in range(3):
        pltpu.async_copy(cost_hbm.at[pl.ds(base + idx * C, C)],
                         bufs[idx], sems[idx])

    def row_quad(k4, carry):
        kb = 4 * k4
        for idx in range(4):
            row = kb + idx
            pltpu.make_async_copy(cost_hbm.at[pl.ds(base + row * C, C)],
                                  bufs[idx], sems[idx]).wait()
            nxt = row + 3
            @pl.when(nxt < ROWS_PER_TILE)
            def _():
                pltpu.async_copy(cost_hbm.at[pl.ds(base + nxt * C, C)],
                                 bufs[(idx + 3) % 4], sems[(idx + 3) % 4])
            scan_row(bufs[idx], row)
        return carry

    lax.fori_loop(0, ROWS_PER_TILE // 4, row_quad, 0)

    pltpu.sync_copy(rbv_loc, rbv_s.at[pl.ds(r0, ROWS_PER_TILE)])
    pltpu.sync_copy(rbc_loc, rbc_s.at[pl.ds(r0, ROWS_PER_TILE)])
    pltpu.make_async_copy(cost_hbm.at[pl.ds(base, SLAB)],
                          mat_s.at[pl.ds(base, SLAB)], sem_e).wait()
    plsc.subcore_barrier()

    @pl.when(sid == 0)
    def _greedy():
        pltpu.sync_copy(rbv_s, rbv)
        pltpu.sync_copy(rbc_s, rbc)
        ones = jnp.full((L,), 1, _i32)

        def zchunk(q, carry):
            colmask[pl.ds(q * L, L)] = jnp.zeros((L,), _f32)
            cnt[pl.ds(q * L, L)] = jnp.zeros((L,), _i32)
            return carry
        lax.fori_loop(0, CCH, zchunk, 0)

        def cmchunk(q, carry):
            hm = _bfly_max(rbv[pl.ds(q * L, L)], iota)
            plsc.store_scatter(cm, [jnp.full((L,), q, _i32)], hm, mask=lane0)
            return carry
        lax.fori_loop(0, RCH, cmchunk, 0)

        def cntchunk(q, carry):
            cb = rbc[pl.ds(q * L, L)]
            for l in range(L):
                plsc.addupdate_scatter(cnt, [cb], ones, mask=iota == l)
            return carry
        lax.fori_loop(0, RCH, cntchunk, 0)

        def step(assigned):
            va = cm[pl.ds(0, L)]
            vb = cm[pl.ds(L, L)]
            tb = vb > va
            m1 = jnp.where(tb, vb, va)
            q1 = jnp.where(tb, iota + L, iota)
            m2 = jnp.where(tb, va, vb)
            q2 = jnp.where(tb, iota, iota + L)
            q1v, g2v, q2v = _bfly_argmax2_full(m1, q1, m2, q2, iota)
            rows1 = q1v * L + iota
            chunk1 = plsc.load_gather(rbv, [rows1])
            r1vec, hm1 = _bfly_argmax_top2(chunk1, rows1, iota)
            rows2 = q2v * L + iota
            chunk2 = plsc.load_gather(rbv, [rows2])
            r2vec, hm2 = _bfly_argmax_top2(chunk2, rows2, iota)
            c1vec = plsc.load_gather(rbc, [r1vec])
            c2vec = plsc.load_gather(rbc, [r2vec])
            plsc.store_scatter(outc, [r1vec], c1vec, mask=lane0)
            plsc.store_scatter(rbv, [r1vec], negvec, mask=lane0)
            plsc.store_scatter(colmask, [c1vec], negvec, mask=lane0)
            plsc.store_scatter(cm, [q1v], hm1, mask=lane0)
            cn1 = _lane0(plsc.load_gather(cnt, [c1vec])) - 1
            ok2v = jnp.where((g2v > hm1) & (c2vec != c1vec), ones, zvec)
            cond2 = (_lane0(ok2v) > 0) & (cn1 == 0)

            @pl.when(cond2)
            def _assign2():
                plsc.store_scatter(outc, [r2vec], c2vec, mask=lane0)
                plsc.store_scatter(rbv, [r2vec], negvec, mask=lane0)
                plsc.store_scatter(colmask, [c2vec], negvec, mask=lane0)
                plsc.store_scatter(cm, [q2v], hm2, mask=lane0)

            cond2v = jnp.full((L,), cond2, jnp.bool_)
            ctgt = jnp.where(cond2v, c2vec, c1vec)

            def dscan():
                def dchunk(q, jminv):
                    cb = rbc[pl.ds(q * L, L)]
                    vb_ = rbv[pl.ds(q * L, L)]
                    match = (cb == ctgt) & (vb_ > negvec)
                    rowv = q * L + iota
                    return jnp.where(match, jnp.minimum(jminv, rowv), jminv)
                jminv = lax.fori_loop(0, RCH, dchunk, bigvec, unroll=8)
                return _lane0(_bfly_min(jminv, iota))

            def rcond(cc):
                return cc > 0

            def rbody(cc):
                j = dscan()
                pltpu.sync_copy(mat_s.at[pl.ds(j * C, C)], rowbuf)
                def rchunk(q, mc):
                    mr, mir = mc
                    v = rowbuf[pl.ds(q * L, L)] + colmask[pl.ds(q * L, L)]
                    colv = q * L + iota
                    better = v > mr
                    return (jnp.where(better, v, mr),
                            jnp.where(better, colv, mir))
                mr, mir = lax.fori_loop(0, CCH, rchunk, (negvec, zvec), unroll=8)
                mxv, cnv = _bfly_argmax(mr, mir, iota)
                jvec = jnp.full((L,), j, _i32)
                plsc.store_scatter(rbv, [jvec], mxv, mask=lane0)
                plsc.store_scatter(rbc, [jvec], cnv, mask=lane0)
                plsc.addupdate_scatter(cnt, [cnv], ones, mask=lane0)
                jq = lax.shift_right_logical(j, 4)
                hmr = _bfly_max(rbv[pl.ds(jq * L, L)], iota)
                plsc.store_scatter(cm, [jnp.full((L,), jq, _i32)], hmr,
                                   mask=lane0)
                return cc - 1

            cc0 = _lane0(plsc.load_gather(cnt, [ctgt])) - 1
            lax.while_loop(rcond, rbody, cc0)
            return assigned + 1 + cond2.astype(_i32)

        lax.while_loop(lambda a: a < R, step, 0)

        def ochunk(q, carry):
            outr[pl.ds(q * L, L)] = q * L + iota
            return carry
        lax.fori_loop(0, RCH, ochunk, 0)
        pltpu.sync_copy(outr, rowind_hbm)
        pltpu.sync_copy(outc, colind_hbm)


_lsa = pl.kernel(
    _lsa_body,
    out_type=(jax.ShapeDtypeStruct((R,), _i32),
              jax.ShapeDtypeStruct((R,), _i32)),
    mesh=plsc.VectorSubcoreMesh(core_axis_name="c", subcore_axis_name="s",
                                num_cores=1, num_subcores=N_TILES),
    compiler_params=pltpu.CompilerParams(needs_layout_passes=False),
    scratch_types=[
        pltpu.VMEM((C,), _f32),
        pltpu.VMEM((C,), _f32),
        pltpu.VMEM((C,), _f32),
        pltpu.VMEM((C,), _f32),
        pltpu.VMEM((C,), _f32),
        pltpu.VMEM((C,), _i32),
        pltpu.VMEM((R,), _f32),
        pltpu.VMEM((R,), _i32),
        pltpu.VMEM((RCH,), _f32),
        pltpu.VMEM((ROWS_PER_TILE,), _f32),
        pltpu.VMEM((ROWS_PER_TILE,), _i32),
        pltpu.VMEM((R,), _i32),
        pltpu.VMEM((R,), _i32),
        pltpu.VMEM_SHARED((R * C,), _f32),
        pltpu.VMEM_SHARED((R,), _f32),
        pltpu.VMEM_SHARED((R,), _i32),
        pltpu.SemaphoreType.DMA,
        pltpu.SemaphoreType.DMA,
        pltpu.SemaphoreType.DMA,
        pltpu.SemaphoreType.DMA,
        pltpu.SemaphoreType.DMA,
    ],
)


def kernel(cost_matrix):
    flat = cost_matrix.reshape(-1)
    row_ind, col_ind = _lsa(flat)
    return row_ind, col_ind


# speedup vs baseline: 85.1177x; 1.1998x over previous
import functools

import jax
import jax.numpy as jnp
from jax import lax
from jax.experimental import pallas as pl
from jax.experimental.pallas import tpu as pltpu
from jax.experimental.pallas import tpu_sc as plsc

R = 512
C = 2048
L = 16
RCH = R // L
CCH = C // L
N_TILES = 16
ROWS_PER_TILE = R // N_TILES
SLAB = ROWS_PER_TILE * C
BIG = 1 << 30

_i32 = jnp.int32
_f32 = jnp.float32


def _bfly_argmax(v, mi, iota):
    for d in (1, 2, 4, 8):
        pidx = iota ^ d
        ov = v.at[pidx].get(mode="promise_in_bounds")
        oi = mi.at[pidx].get(mode="promise_in_bounds")
        take = (ov > v) | ((ov == v) & (oi < mi))
        v = jnp.where(take, ov, v)
        mi = jnp.where(take, oi, mi)
    return v, mi


def _bfly_min(x, iota):
    for d in (1, 2, 4, 8):
        x = jnp.minimum(x, x.at[iota ^ d].get(mode="promise_in_bounds"))
    return x


def _bfly_max(x, iota):
    for d in (1, 2, 4, 8):
        x = jnp.maximum(x, x.at[iota ^ d].get(mode="promise_in_bounds"))
    return x


def _lane0(x):
    return lax.squeeze(lax.slice(x, (0,), (1,)), (0,))


def _bfly_argmax2_full(v, vi, v2, vi2, iota):
    m1, i1, m2, i2 = v, vi, v2, vi2
    for d in (1, 2, 4, 8):
        pidx = iota ^ d
        o1 = m1.at[pidx].get(mode="promise_in_bounds")
        oi1 = i1.at[pidx].get(mode="promise_in_bounds")
        o2 = m2.at[pidx].get(mode="promise_in_bounds")
        oi2 = i2.at[pidx].get(mode="promise_in_bounds")
        t1 = (o1 > m1) | ((o1 == m1) & (oi1 < i1))
        w1 = jnp.where(t1, o1, m1)
        wi1 = jnp.where(t1, oi1, i1)
        l1 = jnp.where(t1, m1, o1)
        li1 = jnp.where(t1, i1, oi1)
        t2 = (o2 > m2) | ((o2 == m2) & (oi2 < i2))
        w2 = jnp.where(t2, o2, m2)
        wi2 = jnp.where(t2, oi2, i2)
        t3 = (l1 > w2) | ((l1 == w2) & (li1 < wi2))
        m1, i1 = w1, wi1
        m2 = jnp.where(t3, l1, w2)
        i2 = jnp.where(t3, li1, wi2)
    return m1, i1, m2, i2


def _top2_insert(m1, i1, m2, i2, v, c):
    t1 = v > m1
    nm1 = jnp.where(t1, v, m1)
    ni1 = jnp.where(t1, c, i1)
    lose = jnp.where(t1, m1, v)
    losei = jnp.where(t1, i1, c)
    t2 = lose > m2
    nm2 = jnp.where(t2, lose, m2)
    ni2 = jnp.where(t2, losei, i2)
    return nm1, ni1, nm2, ni2


def _bfly_argmax_top2(v, mi, iota):
    m1, i1, m2 = v, mi, jnp.full((L,), -jnp.inf, _f32)
    for d in (1, 2, 4, 8):
        pidx = iota ^ d
        o1 = m1.at[pidx].get(mode="promise_in_bounds")
        oi = i1.at[pidx].get(mode="promise_in_bounds")
        o2 = m2.at[pidx].get(mode="promise_in_bounds")
        take = (o1 > m1) | ((o1 == m1) & (oi < i1))
        nm1 = jnp.where(take, o1, m1)
        ni1 = jnp.where(take, oi, i1)
        nm2 = jnp.maximum(jnp.minimum(m1, o1), jnp.maximum(m2, o2))
        m1, i1, m2 = nm1, ni1, nm2
    return i1, m2


def _lsa_body(cost_hbm, rowind_hbm, colind_hbm,
              rowbuf, rowbuf_b, rowbuf_c, rowbuf_d, colmask, cnt, rbv, rbc,
              rbv2, rbc2, cm, rbv_loc, rbc_loc, rbv2_loc, rbc2_loc,
              outr, outc, mat_s, rbv_s, rbc_s, rbv2_s, rbc2_s,
              sem_a, sem_b, sem_c, sem_d, sem_e):
    sid = lax.axis_index("s")
    iota = lax.iota(_i32, L)
    lane0 = iota == 0
    negvec = jnp.full((L,), -jnp.inf, _f32)
    zvec = jnp.zeros((L,), _i32)
    bigvec = jnp.full((L,), BIG, _i32)

    r0 = sid * ROWS_PER_TILE
    base = r0 * C

    pltpu.async_copy(cost_hbm.at[pl.ds(base, SLAB)],
                     mat_s.at[pl.ds(base, SLAB)], sem_e)

    def scan_row(buf, k):
        def chunk(q, mc):
            v = buf[pl.ds(q * L, L)]
            colv = q * L + iota
            return _top2_insert(*mc, v, colv)
        m, mi, m2_, mi2_ = lax.fori_loop(
            0, CCH, chunk, (negvec, zvec, negvec, bigvec), unroll=8)
        mxv, civ, mx2v, ci2v = _bfly_argmax2_full(m, mi, m2_, mi2_, iota)
        kvec = jnp.full((L,), k, _i32)
        plsc.store_scatter(rbv_loc, [kvec], mxv, mask=lane0)
        plsc.store_scatter(rbc_loc, [kvec], civ, mask=lane0)
        plsc.store_scatter(rbv2_loc, [kvec], mx2v, mask=lane0)
        plsc.store_scatter(rbc2_loc, [kvec], ci2v, mask=lane0)

    bufs = (rowbuf, rowbuf_b, rowbuf_c, rowbuf_d)
    sems = (sem_a, sem_b, sem_c, sem_d)
    for idx in range(3):
        pltpu.async_copy(cost_hbm.at[pl.ds(base + idx * C, C)],
                         bufs[idx], sems[idx])

    def row_quad(k4, carry):
        kb = 4 * k4
        for idx in range(4):
            row = kb + idx
            pltpu.make_async_copy(cost_hbm.at[pl.ds(base + row * C, C)],
                                  bufs[idx], sems[idx]).wait()
            nxt = row + 3
            @pl.when(nxt < ROWS_PER_TILE)
            def _():
                pltpu.async_copy(cost_hbm.at[pl.ds(base + nxt * C, C)],
                                 bufs[(idx + 3) % 4], sems[(idx + 3) % 4])
            scan_row(bufs[idx], row)
        return carry

    lax.fori_loop(0, ROWS_PER_TILE // 4, row_quad, 0)

    pltpu.sync_copy(rbv_loc, rbv_s.at[pl.ds(r0, ROWS_PER_TILE)])
    pltpu.sync_copy(rbc_loc, rbc_s.at[pl.ds(r0, ROWS_PER_TILE)])
    pltpu.sync_copy(rbv2_loc, rbv2_s.at[pl.ds(r0, ROWS_PER_TILE)])
    pltpu.sync_copy(rbc2_loc, rbc2_s.at[pl.ds(r0, ROWS_PER_TILE)])
    pltpu.make_async_copy(cost_hbm.at[pl.ds(base, SLAB)],
                          mat_s.at[pl.ds(base, SLAB)], sem_e).wait()
    plsc.subcore_barrier()

    @pl.when(sid == 0)
    def _greedy():
        pltpu.sync_copy(rbv_s, rbv)
        pltpu.sync_copy(rbc_s, rbc)
        pltpu.sync_copy(rbv2_s, rbv2)
        pltpu.sync_copy(rbc2_s, rbc2)
        ones = jnp.full((L,), 1, _i32)

        def zchunk(q, carry):
            colmask[pl.ds(q * L, L)] = jnp.zeros((L,), _f32)
            cnt[pl.ds(q * L, L)] = jnp.zeros((L,), _i32)
            return carry
        lax.fori_loop(0, CCH, zchunk, 0)

        def cmchunk(q, carry):
            hm = _bfly_max(rbv[pl.ds(q * L, L)], iota)
            plsc.store_scatter(cm, [jnp.full((L,), q, _i32)], hm, mask=lane0)
            return carry
        lax.fori_loop(0, RCH, cmchunk, 0)

        def cntchunk(q, carry):
            cb = rbc[pl.ds(q * L, L)]
            for l in range(L):
                plsc.addupdate_scatter(cnt, [cb], ones, mask=iota == l)
            return carry
        lax.fori_loop(0, RCH, cntchunk, 0)

        def step(assigned):
            va = cm[pl.ds(0, L)]
            vb = cm[pl.ds(L, L)]
            tb = vb > va
            m1 = jnp.where(tb, vb, va)
            q1 = jnp.where(tb, iota + L, iota)
            m2 = jnp.where(tb, va, vb)
            q2 = jnp.where(tb, iota, iota + L)
            _, q1v, g2v, q2v = _bfly_argmax2_full(m1, q1, m2, q2, iota)
            rows1 = q1v * L + iota
            chunk1 = plsc.load_gather(rbv, [rows1])
            r1vec, hm1 = _bfly_argmax_top2(chunk1, rows1, iota)
            rows2 = q2v * L + iota
            chunk2 = plsc.load_gather(rbv, [rows2])
            r2vec, hm2 = _bfly_argmax_top2(chunk2, rows2, iota)
            c1vec = plsc.load_gather(rbc, [r1vec])
            c2vec = plsc.load_gather(rbc, [r2vec])
            plsc.store_scatter(outc, [r1vec], c1vec, mask=lane0)
            plsc.store_scatter(rbv, [r1vec], negvec, mask=lane0)
            plsc.store_scatter(colmask, [c1vec], negvec, mask=lane0)
            plsc.store_scatter(cm, [q1v], hm1, mask=lane0)
            cn1 = _lane0(plsc.load_gather(cnt, [c1vec])) - 1
            ok2v = jnp.where((g2v > hm1) & (c2vec != c1vec), ones, zvec)
            cond2 = (_lane0(ok2v) > 0) & (cn1 == 0)

            @pl.when(cond2)
            def _assign2():
                plsc.store_scatter(outc, [r2vec], c2vec, mask=lane0)
                plsc.store_scatter(rbv, [r2vec], negvec, mask=lane0)
                plsc.store_scatter(colmask, [c2vec], negvec, mask=lane0)
                plsc.store_scatter(cm, [q2v], hm2, mask=lane0)

            cond2v = jnp.full((L,), cond2, jnp.bool_)
            ctgt = jnp.where(cond2v, c2vec, c1vec)

            def dscan():
                def dchunk(q, jminv):
                    cb = rbc[pl.ds(q * L, L)]
                    vb_ = rbv[pl.ds(q * L, L)]
                    match = (cb == ctgt) & (vb_ > negvec)
                    rowv = q * L + iota
                    return jnp.where(match, jnp.minimum(jminv, rowv), jminv)
                jminv = lax.fori_loop(0, RCH, dchunk, bigvec, unroll=8)
                return _lane0(_bfly_min(jminv, iota))

            def rcond(cc):
                return cc > 0

            def rbody(cc):
                j = dscan()
                jvec = jnp.full((L,), j, _i32)
                c2j = plsc.load_gather(rbc2, [jvec])
                v2j = plsc.load_gather(rbv2, [jvec])
                c2safe = jnp.maximum(c2j, zvec)
                cmk = plsc.load_gather(colmask, [c2safe])
                okv = jnp.where((c2j >= 0) & (cmk == 0.0), ones, zvec)
                fast = _lane0(okv) > 0

                @pl.when(fast)
                def _fast():
                    plsc.store_scatter(rbv, [jvec], v2j, mask=lane0)
                    plsc.store_scatter(rbc, [jvec], c2j, mask=lane0)
                    plsc.store_scatter(rbc2, [jvec], jnp.full((L,), -1, _i32),
                                       mask=lane0)
                    plsc.addupdate_scatter(cnt, [c2j], ones, mask=lane0)

                @pl.when(jnp.logical_not(fast))
                def _slow():
                    pltpu.sync_copy(mat_s.at[pl.ds(j * C, C)], rowbuf)
                    def rchunk(q, mc):
                        v = rowbuf[pl.ds(q * L, L)] + colmask[pl.ds(q * L, L)]
                        colv = q * L + iota
                        return _top2_insert(*mc, v, colv)
                    mr, mir, mr2, mir2 = lax.fori_loop(
                        0, CCH, rchunk, (negvec, zvec, negvec, bigvec),
                        unroll=8)
                    mxv, cnv, mx2v, cn2v = _bfly_argmax2_full(
                        mr, mir, mr2, mir2, iota)
                    plsc.store_scatter(rbv, [jvec], mxv, mask=lane0)
                    plsc.store_scatter(rbc, [jvec], cnv, mask=lane0)
                    plsc.store_scatter(rbv2, [jvec], mx2v, mask=lane0)
                    plsc.store_scatter(rbc2, [jvec], cn2v, mask=lane0)
                    plsc.addupdate_scatter(cnt, [cnv], ones, mask=lane0)

                jq = lax.shift_right_logical(j, 4)
                hmr = _bfly_max(rbv[pl.ds(jq * L, L)], iota)
                plsc.store_scatter(cm, [jnp.full((L,), jq, _i32)], hmr,
                                   mask=lane0)
                return cc - 1

            cc0 = _lane0(plsc.load_gather(cnt, [ctgt])) - 1
            lax.while_loop(rcond, rbody, cc0)
            return assigned + 1 + cond2.astype(_i32)

        lax.while_loop(lambda a: a < R, step, 0)

        def ochunk(q, carry):
            outr[pl.ds(q * L, L)] = q * L + iota
            return carry
        lax.fori_loop(0, RCH, ochunk, 0)
        pltpu.sync_copy(outr, rowind_hbm)
        pltpu.sync_copy(outc, colind_hbm)


_lsa = pl.kernel(
    _lsa_body,
    out_type=(jax.ShapeDtypeStruct((R,), _i32),
              jax.ShapeDtypeStruct((R,), _i32)),
    mesh=plsc.VectorSubcoreMesh(core_axis_name="c", subcore_axis_name="s",
                                num_cores=1, num_subcores=N_TILES),
    compiler_params=pltpu.CompilerParams(needs_layout_passes=False),
    scratch_types=[
        pltpu.VMEM((C,), _f32),
        pltpu.VMEM((C,), _f32),
        pltpu.VMEM((C,), _f32),
        pltpu.VMEM((C,), _f32),
        pltpu.VMEM((C,), _f32),
        pltpu.VMEM((C,), _i32),
        pltpu.VMEM((R,), _f32),
        pltpu.VMEM((R,), _i32),
        pltpu.VMEM((R,), _f32),
        pltpu.VMEM((R,), _i32),
        pltpu.VMEM((RCH,), _f32),
        pltpu.VMEM((ROWS_PER_TILE,), _f32),
        pltpu.VMEM((ROWS_PER_TILE,), _i32),
        pltpu.VMEM((ROWS_PER_TILE,), _f32),
        pltpu.VMEM((ROWS_PER_TILE,), _i32),
        pltpu.VMEM((R,), _i32),
        pltpu.VMEM((R,), _i32),
        pltpu.VMEM_SHARED((R * C,), _f32),
        pltpu.VMEM_SHARED((R,), _f32),
        pltpu.VMEM_SHARED((R,), _i32),
        pltpu.VMEM_SHARED((R,), _f32),
        pltpu.VMEM_SHARED((R,), _i32),
        pltpu.SemaphoreType.DMA,
        pltpu.SemaphoreType.DMA,
        pltpu.SemaphoreType.DMA,
        pltpu.SemaphoreType.DMA,
        pltpu.SemaphoreType.DMA,
    ],
)


def kernel(cost_matrix):
    flat = cost_matrix.reshape(-1)
    row_ind, col_ind = _lsa(flat)
    return row_ind, col_ind
